# baseline (speedup 1.0000x reference)
import jax
import jax.numpy as jnp
from jax import lax
from jax.experimental import pallas as pl

_B, _D = 16384, 64
_B2, _W = _B // 2, 2 * _D
_BLK = 2048
_NS = _W // 16


def _tc_body(a_ref, b_ref, c_ref, d_ref, s_ref, o_ref):
    p = a_ref[:] * b_ref[:] + c_ref[:] * d_ref[:]
    o_ref[:] = lax.dot_general(s_ref[:], p, (((1,), (1,)), ((), ())),
                               preferred_element_type=jnp.float32)


def kernel(gu, gi, gut, git):
    a = gu.reshape(_B2, _W)
    b = gi.reshape(_B2, _W)
    c = gut.reshape(_B2, _W)
    d = git.reshape(_B2, _W)
    lanes = jnp.arange(_W, dtype=jnp.int32)
    seg = (lanes[None, :] // 16 == jnp.arange(_NS, dtype=jnp.int32)[:, None])
    seg = seg.astype(jnp.float32)
    r = pl.pallas_call(
        _tc_body,
        grid=(_B2 // _BLK,),
        in_specs=[pl.BlockSpec((_BLK, _W), lambda i: (i, 0))] * 4
        + [pl.BlockSpec((_NS, _W), lambda i: (0, 0))],
        out_specs=pl.BlockSpec((_NS, _BLK), lambda i: (0, i)),
        out_shape=jax.ShapeDtypeStruct((_NS, _B2), jnp.float32),
    )(a, b, c, d, seg)
    halves = r.reshape(2, 4, _B2).sum(axis=1)
    return halves.T.reshape(_B)


# speedup vs baseline: 1.6782x; 1.6782x over previous
import jax
import jax.numpy as jnp
from jax import lax
from jax.experimental import pallas as pl

_B, _D = 16384, 64
_BLK = 2048
_NS = 8


def _tc_body(a_ref, b_ref, c_ref, d_ref, s_ref, o_ref):
    p = a_ref[:] * b_ref[:] + c_ref[:] * d_ref[:]
    o_ref[:] = lax.dot_general(s_ref[:], p, (((1,), (1,)), ((), ())),
                               preferred_element_type=jnp.float32)


def kernel(gu, gi, gut, git):
    lanes = jnp.arange(_D, dtype=jnp.int32)
    seg = (lanes[None, :] // 16 == jnp.arange(_NS, dtype=jnp.int32)[:, None])
    seg = seg.astype(jnp.float32)
    r = pl.pallas_call(
        _tc_body,
        grid=(_B // _BLK,),
        in_specs=[pl.BlockSpec((_BLK, _D), lambda i: (i, 0))] * 4
        + [pl.BlockSpec((_NS, _D), lambda i: (0, 0))],
        out_specs=pl.BlockSpec((_NS, _BLK), lambda i: (0, i)),
        out_shape=jax.ShapeDtypeStruct((_NS, _B), jnp.float32),
    )(gu, gi, gut, git, seg)
    return r.sum(axis=0)


# speedup vs baseline: 1.8226x; 1.0860x over previous
import jax
import jax.numpy as jnp
from jax import lax
from jax.experimental import pallas as pl
from jax.experimental.pallas import tpu as pltpu

_B, _D = 16384, 64
_BLK = 2048
_NS = 8


def _tc_body(a_ref, b_ref, c_ref, d_ref, s_ref, o_ref):
    p = a_ref[:] * b_ref[:] + c_ref[:] * d_ref[:]
    o_ref[:] = lax.dot_general(s_ref[:], p, (((1,), (1,)), ((), ())),
                               preferred_element_type=jnp.float32)


def kernel(gu, gi, gut, git):
    lanes = jnp.arange(_D, dtype=jnp.int32)
    seg = (lanes[None, :] // 16 == jnp.arange(_NS, dtype=jnp.int32)[:, None])
    seg = seg.astype(jnp.float32)
    r = pl.pallas_call(
        _tc_body,
        grid=(_B // _BLK,),
        in_specs=[pl.BlockSpec((_BLK, _D), lambda i: (i, 0))] * 4
        + [pl.BlockSpec((_NS, _D), lambda i: (0, 0))],
        out_specs=pl.BlockSpec((_NS, _BLK), lambda i: (0, i)),
        out_shape=jax.ShapeDtypeStruct((_NS, _B), jnp.float32),
        compiler_params=pltpu.CompilerParams(
            allow_input_fusion=[True] * 5),
    )(gu, gi, gut, git, seg)
    return r.sum(axis=0)


# speedup vs baseline: 1.8513x; 1.0158x over previous
import jax
import jax.numpy as jnp
from jax import lax
from jax.experimental import pallas as pl
from jax.experimental.pallas import tpu as pltpu

_B, _D = 16384, 64
_BLK = 4096
_NS = 8


def _tc_body(a_ref, b_ref, c_ref, d_ref, s_ref, o_ref):
    p = a_ref[:] * b_ref[:] + c_ref[:] * d_ref[:]
    o_ref[:] = lax.dot_general(s_ref[:], p, (((1,), (1,)), ((), ())),
                               preferred_element_type=jnp.float32)


def kernel(gu, gi, gut, git):
    lanes = jnp.arange(_D, dtype=jnp.int32)
    seg = (lanes[None, :] // 16 == jnp.arange(_NS, dtype=jnp.int32)[:, None])
    seg = seg.astype(jnp.float32)
    r = pl.pallas_call(
        _tc_body,
        grid=(_B // _BLK,),
        in_specs=[pl.BlockSpec((_BLK, _D), lambda i: (i, 0))] * 4
        + [pl.BlockSpec((_NS, _D), lambda i: (0, 0))],
        out_specs=pl.BlockSpec((_NS, _BLK), lambda i: (0, i)),
        out_shape=jax.ShapeDtypeStruct((_NS, _B), jnp.float32),
        compiler_params=pltpu.CompilerParams(
            allow_input_fusion=[True] * 5),
    )(gu, gi, gut, git, seg)
    return r.sum(axis=0)
